# baseline (speedup 1.0000x reference)
import functools
import jax
import jax.numpy as jnp
from jax import lax
from jax.experimental import pallas as pl
from jax.experimental.pallas import tpu as pltpu
from jax.experimental.pallas import tpu_sc as plsc

_N = 100000
_D = 768
_R = 32
_NT = _N // _R
_NW = 32
_TILES_HI = 98
_SPLIT = _NT - 97 * _NW
_EPS = 1e-8


def _rsqrt(v):
    y = plsc.bitcast(
        jnp.int32(0x5F3759DF) - lax.shift_right_logical(
            plsc.bitcast(v, jnp.int32), jnp.int32(1)), jnp.float32)
    for _ in range(3):
        y = y * (1.5 - 0.5 * v * y * y)
    return y


def _sc_cosine(x_hbm, u_hbm, o_hbm, buf0, buf1, uq, ostage, tmp16, si0, si1, squ):
    cid = lax.axis_index("c")
    sid = lax.axis_index("s")
    wid = sid * 2 + cid
    nt = jnp.where(wid < _SPLIT, _TILES_HI, _TILES_HI - 1)
    t0 = wid * 97 + jnp.minimum(wid, _SPLIT)

    pltpu.sync_copy(u_hbm.at[0], uq)

    un2 = jnp.zeros((16,), jnp.float32)
    for c in range(_D // 16):
        uc = uq[pl.ds(c * 16, 16)]
        un2 = un2 + uc * uc
    lanes = lax.iota(jnp.int32, 16)
    for sh in (1, 2, 4, 8):
        tmp16[...] = un2
        un2 = un2 + plsc.load_gather(tmp16, [lanes ^ sh])
    un2v = un2
    unv = un2v * _rsqrt(jnp.maximum(un2v, 1e-30))
    unv = jnp.maximum(unv, _EPS)

    rows0 = lax.iota(jnp.int32, 16)
    rows1 = rows0 + 16
    zero = jnp.zeros((16,), jnp.float32)
    bufs = (buf0, buf1)
    sis = (si0, si1)

    def in_slice(t):
        return x_hbm.at[pl.ds(t * _R, _R)]

    pltpu.async_copy(in_slice(t0), buf0, si0)

    def outer(i2, _):
        for b in range(2):
            i = i2 * 2 + b
            t = t0 + i

            @pl.when(i < nt)
            def _():
                @pl.when(i + 1 < nt)
                def _():
                    pltpu.async_copy(in_slice(t + 1), bufs[1 - b], sis[1 - b])

                pltpu.make_async_copy(in_slice(t), bufs[b], sis[b]).wait()
                buf = bufs[b]

                def feat(jc, carry):
                    d0, q0, d1, q1 = carry
                    uv = uq[pl.ds(jc * 16, 16)]
                    base = jnp.full((16,), jc * 16, jnp.int32)
                    for l in range(16):
                        jv = base + l
                        v0 = plsc.load_gather(buf, [rows0, jv])
                        v1 = plsc.load_gather(buf, [rows1, jv])
                        us = uv[l]
                        d0 = d0 + v0 * us
                        q0 = q0 + v0 * v0
                        d1 = d1 + v1 * us
                        q1 = q1 + v1 * v1
                    return d0, q0, d1, q1

                d0, q0, d1, q1 = lax.fori_loop(
                    0, _D // 16, feat, (zero, zero, zero, zero), unroll=2)

                for gi, (d, q) in enumerate(((d0, q0), (d1, q1))):
                    xn = q * _rsqrt(jnp.maximum(q, 1e-30))
                    den = unv * jnp.maximum(xn, _EPS)
                    ostage[pl.ds(i * _R + gi * 16, 16)] = d / den
        return 0

    lax.fori_loop(0, (_TILES_HI + 1) // 2, outer, 0)

    @pl.when(nt == _TILES_HI)
    def _():
        pltpu.sync_copy(ostage, o_hbm.at[pl.ds(t0 * _R, _TILES_HI * _R)])

    @pl.when(nt != _TILES_HI)
    def _():
        n = (_TILES_HI - 1) * _R
        pltpu.sync_copy(ostage.at[pl.ds(0, n)], o_hbm.at[pl.ds(t0 * _R, n)])


def kernel(x, user_embed):
    mesh = plsc.VectorSubcoreMesh(core_axis_name="c", subcore_axis_name="s")
    f = functools.partial(
        pl.kernel,
        out_type=jax.ShapeDtypeStruct((_N,), jnp.float32),
        mesh=mesh,
        scratch_types=[
            pltpu.VMEM((_R, _D), jnp.float32),
            pltpu.VMEM((_R, _D), jnp.float32),
            pltpu.VMEM((_D,), jnp.float32),
            pltpu.VMEM((_TILES_HI * _R,), jnp.float32),
            pltpu.VMEM((16,), jnp.float32),
            pltpu.SemaphoreType.DMA,
            pltpu.SemaphoreType.DMA,
            pltpu.SemaphoreType.DMA,
        ],
        compiler_params=pltpu.CompilerParams(needs_layout_passes=False),
    )(_sc_cosine)
    return f(x, user_embed)


# speedup vs baseline: 6.4482x; 6.4482x over previous
import functools
import jax
import jax.numpy as jnp
from jax import lax
from jax.experimental import pallas as pl
from jax.experimental.pallas import tpu as pltpu
from jax.experimental.pallas import tpu_sc as plsc

_N = 100000
_D = 768
_R = 32
_NW = 32
_EPS = 1e-8

_SC_ROWS = _N
assert _SC_ROWS % _R == 0
_NT = _SC_ROWS // _R
_TPW = -(-_NT // _NW)
_REM = _NT % _NW
_LO = _NT // _NW


def _rsqrt(v):
    y = plsc.bitcast(
        jnp.int32(0x5F3759DF) - lax.shift_right_logical(
            plsc.bitcast(v, jnp.int32), jnp.int32(1)), jnp.float32)
    for _ in range(3):
        y = y * (1.5 - 0.5 * v * y * y)
    return y


def _sc_cosine(x_hbm, u_hbm, o_hbm, buf0, buf1, uq, ostage, tpad,
               si0, si1, squ):
    cid = lax.axis_index("c")
    sid = lax.axis_index("s")
    wid = sid * 2 + cid
    if _REM:
        nt = jnp.where(wid < _REM, _TPW, _LO)
        t0 = wid * _LO + jnp.minimum(wid, _REM)
    else:
        nt = jnp.full((), _LO, jnp.int32)
        t0 = wid * _LO

    pltpu.sync_copy(u_hbm.at[0], uq)

    un2 = jnp.zeros((16,), jnp.float32)
    for c in range(_D // 16):
        uc = uq[pl.ds(c * 16, 16)]
        un2 = un2 + uc * uc
    lanes = lax.iota(jnp.int32, 16)
    for sh in (1, 2, 4, 8):
        tpad[0, pl.ds(0, 16)] = un2
        un2 = un2 + plsc.load_gather(tpad, [jnp.zeros((16,), jnp.int32),
                                            lanes ^ sh])
    unv = un2 * _rsqrt(jnp.maximum(un2, 1e-30))
    unv = jnp.maximum(unv, _EPS)

    zero = jnp.zeros((16,), jnp.float32)
    bufs = (buf0, buf1)
    sis = (si0, si1)

    def in_slice(t):
        return x_hbm.at[pl.ds(t * _R, _R)]

    pltpu.async_copy(in_slice(t0), buf0, si0)

    def outer(i2, _):
        for b in range(2):
            i = i2 * 2 + b
            t = t0 + i

            @pl.when(i < nt)
            def _():
                @pl.when(i + 1 < nt)
                def _():
                    pltpu.async_copy(in_slice(t + 1), bufs[1 - b], sis[1 - b])

                pltpu.make_async_copy(in_slice(t), bufs[b], sis[b]).wait()
                buf = bufs[b]

                for g in range(_R // 16):
                    def chunk(c, carry):
                        uc = uq[pl.ds(c * 16, 16)]
                        out = []
                        for r in range(16):
                            v = buf[g * 16 + r, pl.ds(c * 16, 16)]
                            out.append(carry[2 * r] + v * uc)
                            out.append(carry[2 * r + 1] + v * v)
                        return tuple(out)

                    accs = lax.fori_loop(0, _D // 16, chunk,
                                         (zero,) * 32, unroll=2)

                    d = zero
                    q = zero
                    for r in range(16):
                        tpad[r, pl.ds(0, 16)] = accs[2 * r]
                    for c in range(16):
                        cv = jnp.full((16,), c, jnp.int32)
                        d = d + plsc.load_gather(tpad, [lanes, cv])
                    for r in range(16):
                        tpad[r, pl.ds(0, 16)] = accs[2 * r + 1]
                    for c in range(16):
                        cv = jnp.full((16,), c, jnp.int32)
                        q = q + plsc.load_gather(tpad, [lanes, cv])

                    xn = q * _rsqrt(jnp.maximum(q, 1e-30))
                    den = unv * jnp.maximum(xn, _EPS)
                    ostage[pl.ds(i * _R + g * 16, 16)] = d / den
        return 0

    lax.fori_loop(0, (_TPW + 1) // 2, outer, 0)

    if _REM:
        @pl.when(nt == _TPW)
        def _():
            pltpu.sync_copy(ostage, o_hbm.at[pl.ds(t0 * _R, _TPW * _R)])

        @pl.when(nt != _TPW)
        def _():
            pltpu.sync_copy(ostage.at[pl.ds(0, _LO * _R)],
                            o_hbm.at[pl.ds(t0 * _R, _LO * _R)])
    else:
        pltpu.sync_copy(ostage.at[pl.ds(0, _LO * _R)],
                        o_hbm.at[pl.ds(t0 * _R, _LO * _R)])


def kernel(x, user_embed):
    mesh = plsc.VectorSubcoreMesh(core_axis_name="c", subcore_axis_name="s")
    f = functools.partial(
        pl.kernel,
        out_type=jax.ShapeDtypeStruct((_SC_ROWS,), jnp.float32),
        mesh=mesh,
        scratch_types=[
            pltpu.VMEM((_R, _D), jnp.float32),
            pltpu.VMEM((_R, _D), jnp.float32),
            pltpu.VMEM((_D,), jnp.float32),
            pltpu.VMEM((_TPW * _R,), jnp.float32),
            pltpu.VMEM((16, 17), jnp.float32),
            pltpu.SemaphoreType.DMA,
            pltpu.SemaphoreType.DMA,
            pltpu.SemaphoreType.DMA,
        ],
        compiler_params=pltpu.CompilerParams(needs_layout_passes=False),
    )(_sc_cosine)
    return f(x, user_embed)
